# baseline (speedup 1.0000x reference)
import functools

import jax
import jax.numpy as jnp
from jax import lax
from jax.experimental import pallas as pl
from jax.experimental.pallas import tpu as pltpu
from jax.experimental.pallas import tpu_sc as plsc

N = 1048576
NCLS = 21
NCORES = 2
NSUB = 16
LANES = 16
NW = NCORES * NSUB
ROWS_PER_W = N // NW
CHUNK = 2048
NCHUNK = ROWS_PER_W // CHUNK
GROUPS = CHUNK // LANES
UNROLL = 4

_LOG_P = (
    7.0376836292e-2,
    -1.1514610310e-1,
    1.1676998740e-1,
    -1.2420140846e-1,
    1.4249322787e-1,
    -1.6668057665e-1,
    2.0000714765e-1,
    -2.4999993993e-1,
    3.3333331174e-1,
)
_SQRTH = 0.70710678118654752440
_LOG_C1 = -2.12194440e-4
_LOG_C2 = 0.693359375


def _vlog(p):
    bits = plsc.bitcast(p, jnp.int32)
    e = (bits >> 23) - 127
    mbits = (bits & 0x007FFFFF) | 0x3F800000
    m = plsc.bitcast(mbits, jnp.float32)
    ef = e.astype(jnp.float32)
    x0 = m * 0.5
    cond = x0 < _SQRTH
    x = jnp.where(cond, m - 1.0, x0 - 1.0)
    en = jnp.where(cond, ef, ef + 1.0)
    z = x * x
    y = jnp.full_like(x, _LOG_P[0])
    for c in _LOG_P[1:]:
        y = y * x + c
    y = x * z * y
    y = y + en * _LOG_C1
    y = y - 0.5 * z
    r = x + y
    return r + en * _LOG_C2


def _sc_body(rin_hbm, lab_hbm, sums_out, cnts_out,
             vbuf0, lbuf0, sums_t, cnts_t, sv0, sl0):
    wid = lax.axis_index("s") * NCORES + lax.axis_index("c")
    row0 = wid * ROWS_PER_W

    z16 = jnp.zeros((LANES,), jnp.float32)
    for c in range(NCLS):
        sums_t[c, :] = z16
        cnts_t[c, :] = z16

    lane = lax.iota(jnp.int32, LANES)
    ones = jnp.ones((LANES,), jnp.float32)

    def chunk_body(k, _):
        base = row0 + k * CHUNK
        pltpu.async_copy(rin_hbm.at[pl.ds(base, CHUNK), :], vbuf0, sv0).wait()
        pltpu.async_copy(lab_hbm.at[pl.ds(base, CHUNK)], lbuf0, sl0).wait()

        @plsc.parallel_loop(0, GROUPS, unroll=UNROLL)
        def _group(g):
            off = g * LANES
            labv = lbuf0[pl.ds(off, LANES)]
            rows = off + lane
            vals = plsc.load_gather(vbuf0, [rows, labv])
            lnp = _vlog(vals)
            plsc.addupdate_scatter(sums_t, [labv, lane], lnp)
            plsc.addupdate_scatter(cnts_t, [labv, lane], ones)

        return 0

    lax.fori_loop(0, NCHUNK, chunk_body, 0)

    pltpu.sync_copy(sums_t, sums_out.at[wid])
    pltpu.sync_copy(cnts_t, cnts_out.at[wid])


_sc_kernel = functools.partial(
    pl.kernel,
    out_type=(
        jax.ShapeDtypeStruct((NW, NCLS, LANES), jnp.float32),
        jax.ShapeDtypeStruct((NW, NCLS, LANES), jnp.float32),
    ),
    mesh=plsc.VectorSubcoreMesh(
        core_axis_name="c", subcore_axis_name="s",
        num_cores=NCORES, num_subcores=NSUB),
    compiler_params=pltpu.CompilerParams(
        needs_layout_passes=False, use_tc_tiling_on_sc=False),
    scratch_types=(
        pltpu.VMEM((CHUNK, NCLS), jnp.float32),
        pltpu.VMEM((CHUNK,), jnp.int32),
        pltpu.VMEM((NCLS, LANES), jnp.float32),
        pltpu.VMEM((NCLS, LANES), jnp.float32),
        pltpu.SemaphoreType.DMA,
        pltpu.SemaphoreType.DMA,
    ),
)(_sc_body)


def _combine_body(sums_ref, cnts_ref, wl_ref, ls_ref, out_ref):
    s = jnp.sum(sums_ref[...], axis=0)
    c = jnp.sum(cnts_ref[...], axis=0)
    per_class = -jnp.sum(s, axis=1, keepdims=True)
    counts = jnp.sum(c, axis=1, keepdims=True)
    present = (counts > 0.0).astype(jnp.float32)
    contrib = wl_ref[...] * (per_class[1:] + 1.0) * present[1:]
    out_ref[...] = jnp.reshape(jnp.sum(contrib) / jnp.sum(ls_ref[...]), (1, 1))


def kernel(realinput, reallabel, Wl, label_sum):
    sums, cnts = _sc_kernel(realinput, reallabel)
    out = pl.pallas_call(
        _combine_body,
        out_shape=jax.ShapeDtypeStruct((1, 1), jnp.float32),
    )(sums, cnts, Wl.reshape(NCLS - 1, 1), label_sum.reshape(NCLS - 1, 1))
    return out[0, 0]


# speedup vs baseline: 2.0154x; 2.0154x over previous
import functools

import jax
import jax.numpy as jnp
from jax import lax
from jax.experimental import pallas as pl
from jax.experimental.pallas import tpu as pltpu
from jax.experimental.pallas import tpu_sc as plsc

N = 1048576
NCLS = 21
NCORES = 2
NSUB = 16
LANES = 16
NW = NCORES * NSUB
ROWS_PER_W = N // NW
CHUNK = 128
NBUF = 4
NCHUNK = ROWS_PER_W // CHUNK
GROUPS = CHUNK // LANES
UNROLL = 4

_LOG_P = (
    7.0376836292e-2,
    -1.1514610310e-1,
    1.1676998740e-1,
    -1.2420140846e-1,
    1.4249322787e-1,
    -1.6668057665e-1,
    2.0000714765e-1,
    -2.4999993993e-1,
    3.3333331174e-1,
)
_SQRTH = 0.70710678118654752440
_LOG_C1 = -2.12194440e-4
_LOG_C2 = 0.693359375


def _vlog(p):
    bits = plsc.bitcast(p, jnp.int32)
    e = (bits >> 23) - 127
    mbits = (bits & 0x007FFFFF) | 0x3F800000
    m = plsc.bitcast(mbits, jnp.float32)
    ef = e.astype(jnp.float32)
    x0 = m * 0.5
    cond = x0 < _SQRTH
    x = jnp.where(cond, m - 1.0, x0 - 1.0)
    en = jnp.where(cond, ef, ef + 1.0)
    z = x * x
    y = jnp.full_like(x, _LOG_P[0])
    for c in _LOG_P[1:]:
        y = y * x + c
    y = x * z * y
    y = y + en * _LOG_C1
    y = y - 0.5 * z
    r = x + y
    return r + en * _LOG_C2


def _sc_body(rin_hbm, lab_hbm, sums_out, cnts_out,
             vb0, vb1, vb2, vb3, lb0, lb1, lb2, lb3,
             sums_t, cnts_t, sv0, sv1, sv2, sv3, sl0, sl1, sl2, sl3):
    wid = lax.axis_index("s") * NCORES + lax.axis_index("c")
    row0 = wid * ROWS_PER_W

    z16 = jnp.zeros((LANES,), jnp.float32)
    for c in range(NCLS):
        sums_t[c, :] = z16
        cnts_t[c, :] = z16

    lane = lax.iota(jnp.int32, LANES)
    ones = jnp.ones((LANES,), jnp.float32)

    vbufs = (vb0, vb1, vb2, vb3)
    lbufs = (lb0, lb1, lb2, lb3)
    svs = (sv0, sv1, sv2, sv3)
    sls = (sl0, sl1, sl2, sl3)

    def start(k, b):
        base = row0 + k * CHUNK
        pltpu.async_copy(
            rin_hbm.at[pl.ds(base, CHUNK), :], vbufs[b], svs[b])
        pltpu.async_copy(lab_hbm.at[pl.ds(base, CHUNK)], lbufs[b], sls[b])

    def wait(b):
        pltpu.make_async_copy(
            rin_hbm.at[pl.ds(0, CHUNK), :], vbufs[b], svs[b]).wait()
        pltpu.make_async_copy(
            lab_hbm.at[pl.ds(0, CHUNK)], lbufs[b], sls[b]).wait()

    for b in range(NBUF - 1):
        start(b, b)

    def ring_body(j, _):
        for b in range(NBUF):
            k = j * NBUF + b

            @pl.when(k + NBUF - 1 < NCHUNK)
            def _():
                start(k + NBUF - 1, (b + NBUF - 1) % NBUF)

            wait(b)
            vbuf = vbufs[b]
            lbuf = lbufs[b]

            @plsc.parallel_loop(0, GROUPS, unroll=UNROLL)
            def _group(g):
                off = g * LANES
                labv = lbuf[pl.ds(off, LANES)]
                rows = off + lane
                vals = plsc.load_gather(vbuf, [rows, labv])
                lnp = _vlog(vals)
                plsc.addupdate_scatter(sums_t, [labv, lane], lnp)
                plsc.addupdate_scatter(cnts_t, [labv, lane], ones)

        return 0

    lax.fori_loop(0, NCHUNK // NBUF, ring_body, 0)

    pltpu.sync_copy(sums_t, sums_out.at[wid])
    pltpu.sync_copy(cnts_t, cnts_out.at[wid])


_sc_kernel = functools.partial(
    pl.kernel,
    out_type=(
        jax.ShapeDtypeStruct((NW, NCLS, LANES), jnp.float32),
        jax.ShapeDtypeStruct((NW, NCLS, LANES), jnp.float32),
    ),
    mesh=plsc.VectorSubcoreMesh(
        core_axis_name="c", subcore_axis_name="s",
        num_cores=NCORES, num_subcores=NSUB),
    compiler_params=pltpu.CompilerParams(needs_layout_passes=False),
    scratch_types=(
        pltpu.VMEM((CHUNK, NCLS), jnp.float32),
        pltpu.VMEM((CHUNK, NCLS), jnp.float32),
        pltpu.VMEM((CHUNK, NCLS), jnp.float32),
        pltpu.VMEM((CHUNK, NCLS), jnp.float32),
        pltpu.VMEM((CHUNK,), jnp.int32),
        pltpu.VMEM((CHUNK,), jnp.int32),
        pltpu.VMEM((CHUNK,), jnp.int32),
        pltpu.VMEM((CHUNK,), jnp.int32),
        pltpu.VMEM((NCLS, LANES), jnp.float32),
        pltpu.VMEM((NCLS, LANES), jnp.float32),
        pltpu.SemaphoreType.DMA,
        pltpu.SemaphoreType.DMA,
        pltpu.SemaphoreType.DMA,
        pltpu.SemaphoreType.DMA,
        pltpu.SemaphoreType.DMA,
        pltpu.SemaphoreType.DMA,
        pltpu.SemaphoreType.DMA,
        pltpu.SemaphoreType.DMA,
    ),
)(_sc_body)


def _combine_body(sums_ref, cnts_ref, wl_ref, ls_ref, out_ref):
    s = jnp.sum(sums_ref[...], axis=0)
    c = jnp.sum(cnts_ref[...], axis=0)
    per_class = -jnp.sum(s, axis=1, keepdims=True)
    counts = jnp.sum(c, axis=1, keepdims=True)
    present = (counts > 0.0).astype(jnp.float32)
    contrib = wl_ref[...] * (per_class[1:] + 1.0) * present[1:]
    out_ref[...] = jnp.reshape(jnp.sum(contrib) / jnp.sum(ls_ref[...]), (1, 1))


def kernel(realinput, reallabel, Wl, label_sum):
    sums, cnts = _sc_kernel(realinput, reallabel)
    out = pl.pallas_call(
        _combine_body,
        out_shape=jax.ShapeDtypeStruct((1, 1), jnp.float32),
    )(sums, cnts, Wl.reshape(NCLS - 1, 1), label_sum.reshape(NCLS - 1, 1))
    return out[0, 0]


# speedup vs baseline: 16.5119x; 8.1928x over previous
import functools

import jax
import jax.numpy as jnp
from jax import lax
from jax.experimental import pallas as pl
from jax.experimental.pallas import tpu as pltpu
from jax.experimental.pallas import tpu_sc as plsc

N = 1048576
NCLS = 21
NCORES = 2
NSUB = 16
LANES = 16
NW = NCORES * NSUB
ROWS_PER_W = N // NW
CHUNK = 1024
NBUF = 4
NCHUNK = ROWS_PER_W // CHUNK
GROUPS = CHUNK // LANES
UNROLL = 4

_LOG_P = (
    7.0376836292e-2,
    -1.1514610310e-1,
    1.1676998740e-1,
    -1.2420140846e-1,
    1.4249322787e-1,
    -1.6668057665e-1,
    2.0000714765e-1,
    -2.4999993993e-1,
    3.3333331174e-1,
)
_SQRTH = 0.70710678118654752440
_LOG_C1 = -2.12194440e-4
_LOG_C2 = 0.693359375


def _vlog(p):
    bits = plsc.bitcast(p, jnp.int32)
    e = (bits >> 23) - 127
    mbits = (bits & 0x007FFFFF) | 0x3F800000
    m = plsc.bitcast(mbits, jnp.float32)
    ef = e.astype(jnp.float32)
    x0 = m * 0.5
    cond = x0 < _SQRTH
    x = jnp.where(cond, m - 1.0, x0 - 1.0)
    en = jnp.where(cond, ef, ef + 1.0)
    z = x * x
    y = jnp.full_like(x, _LOG_P[0])
    for c in _LOG_P[1:]:
        y = y * x + c
    y = x * z * y
    y = y + en * _LOG_C1
    y = y - 0.5 * z
    r = x + y
    return r + en * _LOG_C2


def _sc_body(rt_hbm, lab_hbm, sums_out, cnts_out,
             vb0, vb1, vb2, vb3, lb0, lb1, lb2, lb3,
             sums_t, cnts_t, sv0, sv1, sv2, sv3, sl0, sl1, sl2, sl3):
    wid = lax.axis_index("s") * NCORES + lax.axis_index("c")
    row0 = wid * ROWS_PER_W

    z16 = jnp.zeros((LANES,), jnp.float32)
    for c in range(NCLS):
        sums_t[c, :] = z16
        cnts_t[c, :] = z16

    lane = lax.iota(jnp.int32, LANES)
    ones = jnp.ones((LANES,), jnp.float32)

    vbufs = (vb0, vb1, vb2, vb3)
    lbufs = (lb0, lb1, lb2, lb3)
    svs = (sv0, sv1, sv2, sv3)
    sls = (sl0, sl1, sl2, sl3)

    def start(k, b):
        base = row0 + k * CHUNK
        pltpu.async_copy(
            rt_hbm.at[:, pl.ds(base, CHUNK)], vbufs[b], svs[b])
        pltpu.async_copy(lab_hbm.at[pl.ds(base, CHUNK)], lbufs[b], sls[b])

    def wait(b):
        pltpu.make_async_copy(
            rt_hbm.at[:, pl.ds(0, CHUNK)], vbufs[b], svs[b]).wait()
        pltpu.make_async_copy(
            lab_hbm.at[pl.ds(0, CHUNK)], lbufs[b], sls[b]).wait()

    for b in range(NBUF - 1):
        start(b, b)

    def ring_body(j, _):
        for b in range(NBUF):
            k = j * NBUF + b

            @pl.when(k + NBUF - 1 < NCHUNK)
            def _():
                start(k + NBUF - 1, (b + NBUF - 1) % NBUF)

            wait(b)
            vbuf = vbufs[b]
            lbuf = lbufs[b]

            @plsc.parallel_loop(0, GROUPS, unroll=UNROLL)
            def _group(g):
                off = g * LANES
                labv = lbuf[pl.ds(off, LANES)]
                cols = off + lane
                vals = plsc.load_gather(vbuf, [labv, cols])
                lnp = _vlog(vals)
                plsc.addupdate_scatter(sums_t, [labv, lane], lnp)
                plsc.addupdate_scatter(cnts_t, [labv, lane], ones)

        return 0

    lax.fori_loop(0, NCHUNK // NBUF, ring_body, 0)

    pltpu.sync_copy(sums_t, sums_out.at[wid])
    pltpu.sync_copy(cnts_t, cnts_out.at[wid])


_sc_kernel = functools.partial(
    pl.kernel,
    out_type=(
        jax.ShapeDtypeStruct((NW, NCLS, LANES), jnp.float32),
        jax.ShapeDtypeStruct((NW, NCLS, LANES), jnp.float32),
    ),
    mesh=plsc.VectorSubcoreMesh(
        core_axis_name="c", subcore_axis_name="s",
        num_cores=NCORES, num_subcores=NSUB),
    compiler_params=pltpu.CompilerParams(needs_layout_passes=False),
    scratch_types=(
        pltpu.VMEM((NCLS, CHUNK), jnp.float32),
        pltpu.VMEM((NCLS, CHUNK), jnp.float32),
        pltpu.VMEM((NCLS, CHUNK), jnp.float32),
        pltpu.VMEM((NCLS, CHUNK), jnp.float32),
        pltpu.VMEM((CHUNK,), jnp.int32),
        pltpu.VMEM((CHUNK,), jnp.int32),
        pltpu.VMEM((CHUNK,), jnp.int32),
        pltpu.VMEM((CHUNK,), jnp.int32),
        pltpu.VMEM((NCLS, LANES), jnp.float32),
        pltpu.VMEM((NCLS, LANES), jnp.float32),
        pltpu.SemaphoreType.DMA,
        pltpu.SemaphoreType.DMA,
        pltpu.SemaphoreType.DMA,
        pltpu.SemaphoreType.DMA,
        pltpu.SemaphoreType.DMA,
        pltpu.SemaphoreType.DMA,
        pltpu.SemaphoreType.DMA,
        pltpu.SemaphoreType.DMA,
    ),
)(_sc_body)


def _combine_body(sums_ref, cnts_ref, wl_ref, ls_ref, out_ref):
    s = jnp.sum(sums_ref[...], axis=0)
    c = jnp.sum(cnts_ref[...], axis=0)
    per_class = -jnp.sum(s, axis=1, keepdims=True)
    counts = jnp.sum(c, axis=1, keepdims=True)
    present = (counts > 0.0).astype(jnp.float32)
    contrib = wl_ref[...] * (per_class[1:] + 1.0) * present[1:]
    out_ref[...] = jnp.reshape(jnp.sum(contrib) / jnp.sum(ls_ref[...]), (1, 1))


def kernel(realinput, reallabel, Wl, label_sum):
    sums, cnts = _sc_kernel(realinput.T, reallabel)
    out = pl.pallas_call(
        _combine_body,
        out_shape=jax.ShapeDtypeStruct((1, 1), jnp.float32),
    )(sums, cnts, Wl.reshape(NCLS - 1, 1), label_sum.reshape(NCLS - 1, 1))
    return out[0, 0]


# speedup vs baseline: 16.6700x; 1.0096x over previous
import functools

import jax
import jax.numpy as jnp
from jax import lax
from jax.experimental import pallas as pl
from jax.experimental.pallas import tpu as pltpu
from jax.experimental.pallas import tpu_sc as plsc

N = 1048576
NCLS = 21
NCORES = 2
NSUB = 16
LANES = 16
NW = NCORES * NSUB
ROWS_PER_W = N // NW
CHUNK = 1024
NBUF = 4
NCHUNK = ROWS_PER_W // CHUNK
GROUPS = CHUNK // LANES
UNROLL = 4

_LN2 = 0.6931471805599453


def _vlog(p):
    bits = plsc.bitcast(p, jnp.int32)
    e = (bits >> 23) - 127
    mbits = (bits & 0x007FFFFF) | 0x3F800000
    m = plsc.bitcast(mbits, jnp.float32)
    ef = e.astype(jnp.float32)
    t = (m - 1.0) / (m + 1.0)
    t2 = t * t
    q = (2.0 / 7.0) * t2 + (2.0 / 5.0)
    q = q * t2 + (2.0 / 3.0)
    q = q * t2 + 2.0
    return t * q + ef * _LN2


def _sc_body(rt_hbm, lab_hbm, sums_out, cnts_out,
             vb0, vb1, vb2, vb3, lb0, lb1, lb2, lb3,
             sums_t, cnts_t, sv0, sv1, sv2, sv3, sl0, sl1, sl2, sl3):
    wid = lax.axis_index("s") * NCORES + lax.axis_index("c")
    row0 = wid * ROWS_PER_W

    z16 = jnp.zeros((LANES,), jnp.float32)
    for c in range(NCLS):
        sums_t[c, :] = z16
        cnts_t[c, :] = z16

    lane = lax.iota(jnp.int32, LANES)
    ones = jnp.ones((LANES,), jnp.float32)

    vbufs = (vb0, vb1, vb2, vb3)
    lbufs = (lb0, lb1, lb2, lb3)
    svs = (sv0, sv1, sv2, sv3)
    sls = (sl0, sl1, sl2, sl3)

    def start(k, b):
        base = row0 + k * CHUNK
        pltpu.async_copy(
            rt_hbm.at[:, pl.ds(base, CHUNK)], vbufs[b], svs[b])
        pltpu.async_copy(lab_hbm.at[pl.ds(base, CHUNK)], lbufs[b], sls[b])

    def wait(b):
        pltpu.make_async_copy(
            rt_hbm.at[:, pl.ds(0, CHUNK)], vbufs[b], svs[b]).wait()
        pltpu.make_async_copy(
            lab_hbm.at[pl.ds(0, CHUNK)], lbufs[b], sls[b]).wait()

    for b in range(NBUF - 1):
        start(b, b)

    def ring_body(j, _):
        for b in range(NBUF):
            k = j * NBUF + b

            @pl.when(k + NBUF - 1 < NCHUNK)
            def _():
                start(k + NBUF - 1, (b + NBUF - 1) % NBUF)

            wait(b)
            vbuf = vbufs[b]
            lbuf = lbufs[b]

            @plsc.parallel_loop(0, GROUPS, unroll=UNROLL)
            def _group(g):
                off = g * LANES
                labv = lbuf[pl.ds(off, LANES)]
                cols = off + lane
                vals = plsc.load_gather(vbuf, [labv, cols])
                lnp = _vlog(vals)
                plsc.addupdate_scatter(sums_t, [labv, lane], lnp)
                plsc.addupdate_scatter(cnts_t, [labv, lane], ones)

        return 0

    lax.fori_loop(0, NCHUNK // NBUF, ring_body, 0)

    pltpu.sync_copy(sums_t, sums_out.at[wid])
    pltpu.sync_copy(cnts_t, cnts_out.at[wid])


_sc_kernel = functools.partial(
    pl.kernel,
    out_type=(
        jax.ShapeDtypeStruct((NW, NCLS, LANES), jnp.float32),
        jax.ShapeDtypeStruct((NW, NCLS, LANES), jnp.float32),
    ),
    mesh=plsc.VectorSubcoreMesh(
        core_axis_name="c", subcore_axis_name="s",
        num_cores=NCORES, num_subcores=NSUB),
    compiler_params=pltpu.CompilerParams(needs_layout_passes=False),
    scratch_types=(
        pltpu.VMEM((NCLS, CHUNK), jnp.float32),
        pltpu.VMEM((NCLS, CHUNK), jnp.float32),
        pltpu.VMEM((NCLS, CHUNK), jnp.float32),
        pltpu.VMEM((NCLS, CHUNK), jnp.float32),
        pltpu.VMEM((CHUNK,), jnp.int32),
        pltpu.VMEM((CHUNK,), jnp.int32),
        pltpu.VMEM((CHUNK,), jnp.int32),
        pltpu.VMEM((CHUNK,), jnp.int32),
        pltpu.VMEM((NCLS, LANES), jnp.float32),
        pltpu.VMEM((NCLS, LANES), jnp.float32),
        pltpu.SemaphoreType.DMA,
        pltpu.SemaphoreType.DMA,
        pltpu.SemaphoreType.DMA,
        pltpu.SemaphoreType.DMA,
        pltpu.SemaphoreType.DMA,
        pltpu.SemaphoreType.DMA,
        pltpu.SemaphoreType.DMA,
        pltpu.SemaphoreType.DMA,
    ),
)(_sc_body)


def _combine_body(sums_ref, cnts_ref, wl_ref, ls_ref, out_ref):
    s = jnp.sum(sums_ref[...], axis=0)
    c = jnp.sum(cnts_ref[...], axis=0)
    per_class = -jnp.sum(s, axis=1, keepdims=True)
    counts = jnp.sum(c, axis=1, keepdims=True)
    present = (counts > 0.0).astype(jnp.float32)
    contrib = wl_ref[...] * (per_class[1:] + 1.0) * present[1:]
    out_ref[...] = jnp.reshape(jnp.sum(contrib) / jnp.sum(ls_ref[...]), (1, 1))


def kernel(realinput, reallabel, Wl, label_sum):
    sums, cnts = _sc_kernel(realinput.T, reallabel)
    out = pl.pallas_call(
        _combine_body,
        out_shape=jax.ShapeDtypeStruct((1, 1), jnp.float32),
    )(sums, cnts, Wl.reshape(NCLS - 1, 1), label_sum.reshape(NCLS - 1, 1))
    return out[0, 0]
